# baseline (speedup 1.0000x reference)
import jax
import jax.numpy as jnp
from jax import lax
from jax.experimental import pallas as pl
from jax.experimental.pallas import tpu as pltpu
from jax.experimental.pallas import tpu_sc as plsc

NC = 2
NS = 16
L = 16
NW = NC * NS

N_NODES = 10000
N_EDGES = 320000
D = 128
DJ = D // L

OUT_LO = 1000
OUT_N = 5000

EPW = N_EDGES // NW
CHUNK = 80
NCHUNK = EPW // CHUNK
GPC = CHUNK // L

ACC_ROWS = 5120
RPT = ACC_ROWS // NS

TH_BLK = 128
TH_N = 79 * TH_BLK

_DNUMS = lax.GatherDimensionNumbers(
    offset_dims=(), collapsed_slice_dims=(0,), start_index_map=(0,))


def _lane_bcast(vec, l):
    idx = jnp.full((L, 1), l, jnp.int32)
    return lax.gather(vec, idx, _DNUMS, slice_sizes=(1,),
                      mode=lax.GatherScatterMode.PROMISE_IN_BOUNDS)


_mesh = plsc.VectorSubcoreMesh(
    core_axis_name="c", subcore_axis_name="s", num_cores=NC, num_subcores=NS
)


def _sc_body(interval_hbm, emb_hbm, row_hbm, col_hbm, theta_hbm, out_hbm,
             col_all, row_all, int_all, th0, th1,
             in0, in1, out0, out1, dst0, dst1,
             acc_sh, gsem0, gsem1, ssem0, ssem1):
    c = lax.axis_index("c")
    s = lax.axis_index("s")
    wid = s * NC + c
    iota16 = lax.iota(jnp.int32, L)
    ins = (in0, in1)
    outs = (out0, out1)
    dsts = (dst0, dst1)
    ths = (th0, th1)
    gsems = (gsem0, gsem1)
    ssems = (ssem0, ssem1)

    def _zb_zero(r, _):
        for j in range(DJ):
            out0[r, pl.ds(j * L, L)] = jnp.zeros((L,), jnp.float32)
        return 0

    lax.fori_loop(0, CHUNK, _zb_zero, 0)
    off0 = s * RPT
    for q in range(RPT // CHUNK):
        pltpu.sync_copy(out0, acc_sh.at[pl.ds(off0 + q * CHUNK, CHUNK)])

    ebase = wid * EPW
    pltpu.sync_copy(col_hbm.at[pl.ds(ebase, EPW)], col_all)
    pltpu.sync_copy(row_hbm.at[pl.ds(ebase, EPW)], row_all)
    pltpu.sync_copy(interval_hbm.at[pl.ds(ebase, EPW)], int_all)

    plsc.subcore_barrier()

    def _gather_start(ci, b):
        idx = col_all.at[pl.ds(ci * CHUNK, CHUNK)]
        pltpu.async_copy(emb_hbm.at[idx], ins[b], gsems[b])
        pltpu.async_copy(theta_hbm.at[idx], ths[b], gsems[b])

    def _gather_wait(b):
        idx = col_all.at[pl.ds(0, CHUNK)]
        pltpu.make_async_copy(emb_hbm.at[idx], ins[b], gsems[b]).wait()
        pltpu.make_async_copy(theta_hbm.at[idx], ths[b], gsems[b]).wait()

    def _scatter_start(b):
        pltpu.async_copy(outs[b], acc_sh.at[dsts[b]], ssems[b], add=True)

    def _scatter_wait(b):
        pltpu.make_async_copy(outs[b], acc_sh.at[dsts[b]], ssems[b]).wait()

    def _compute(ci, b):
        base = ci * CHUNK
        ib, ob, db = ins[b], outs[b], dsts[b]

        def _group(g, _):
            off = base + g * L
            int16 = int_all[pl.ds(off, L)]
            row16 = row_all[pl.ds(off, L)]
            ok = (row16 >= OUT_LO) & (row16 < OUT_LO + OUT_N)
            db[pl.ds(g * L, L)] = jnp.where(ok, row16 - OUT_LO, OUT_N + iota16)
            th16 = ths[b][pl.ds(g * L, L)]
            d16 = jnp.exp(int16 * th16)
            for l in range(L):
                e = g * L + l
                dl = _lane_bcast(d16, l)
                for j in range(DJ):
                    ob[e, pl.ds(j * L, L)] = ib[e, pl.ds(j * L, L)] * dl
            return 0

        lax.fori_loop(0, GPC, _group, 0)

    for b in range(2):
        for g in range(GPC):
            dsts[b][pl.ds(g * L, L)] = OUT_N + iota16
        _scatter_start(b)
    _gather_start(0, 0)
    _gather_start(1, 1)

    def _pair(k, _):
        for b in range(2):
            ci = 2 * k + b
            _gather_wait(b)
            _scatter_wait(b)
            _compute(ci, b)
            _scatter_start(b)

            @pl.when(ci < NCHUNK - 2)
            def _():
                _gather_start(ci + 2, b)
        return 0

    lax.fori_loop(0, (NCHUNK - 1) // 2, _pair, 0)

    _gather_wait(0)
    _scatter_wait(0)
    _compute(NCHUNK - 1, 0)
    _scatter_start(0)
    _scatter_wait(1)
    _scatter_wait(0)

    plsc.subcore_barrier()

    for q in range(RPT // CHUNK):
        pltpu.sync_copy(acc_sh.at[pl.ds(off0 + q * CHUNK, CHUNK)], in0)
        pltpu.sync_copy(in0, out_hbm.at[c, pl.ds(off0 + q * CHUNK, CHUNK)])


_sc_kernel = pl.kernel(
    _sc_body,
    out_type=jax.ShapeDtypeStruct((NC, ACC_ROWS, D), jnp.float32),
    mesh=_mesh,
    scratch_types=[
        pltpu.VMEM((EPW,), jnp.int32),
        pltpu.VMEM((EPW,), jnp.int32),
        pltpu.VMEM((EPW,), jnp.float32),
        pltpu.VMEM((CHUNK,), jnp.float32),
        pltpu.VMEM((CHUNK,), jnp.float32),
        pltpu.VMEM((CHUNK, D), jnp.float32),
        pltpu.VMEM((CHUNK, D), jnp.float32),
        pltpu.VMEM((CHUNK, D), jnp.float32),
        pltpu.VMEM((CHUNK, D), jnp.float32),
        pltpu.VMEM((CHUNK,), jnp.int32),
        pltpu.VMEM((CHUNK,), jnp.int32),
        pltpu.VMEM_SHARED((ACC_ROWS, D), jnp.float32),
        pltpu.SemaphoreType.DMA,
        pltpu.SemaphoreType.DMA,
        pltpu.SemaphoreType.DMA,
        pltpu.SemaphoreType.DMA,
    ],
)


def _theta_body(x_ref, p_ref, o_ref):
    o_ref[...] = jnp.sum(x_ref[...] * p_ref[...], axis=1)


_theta_tc = pl.pallas_call(
    _theta_body,
    out_shape=jax.ShapeDtypeStruct((TH_N,), jnp.float32),
    grid=(TH_N // TH_BLK,),
    in_specs=[pl.BlockSpec((TH_BLK, D), lambda i: (i, 0)),
              pl.BlockSpec((1, D), lambda i: (0, 0))],
    out_specs=pl.BlockSpec((TH_BLK,), lambda i: (i,)),
)


def _combine_body(p_ref, o_ref):
    o_ref[...] = p_ref[0] + p_ref[1]


_combine = pl.pallas_call(
    _combine_body,
    out_shape=jax.ShapeDtypeStruct((OUT_N, D), jnp.float32),
    grid=(5,),
    in_specs=[pl.BlockSpec((2, 1000, D), lambda i: (0, i, 0))],
    out_specs=pl.BlockSpec((1000, D), lambda i: (i, 0)),
)


def kernel(interval, embedding, edge_index, params):
    row = edge_index[0]
    col = edge_index[1]
    theta = _theta_tc(embedding, params.reshape(1, D))
    partial = _sc_kernel(interval, embedding, row, col, theta)
    return _combine(partial)


# speedup vs baseline: 1.3559x; 1.3559x over previous
import jax
import jax.numpy as jnp
from jax import lax
from jax.experimental import pallas as pl
from jax.experimental.pallas import tpu as pltpu
from jax.experimental.pallas import tpu_sc as plsc

NC = 2
NS = 16
L = 16
NW = NC * NS

N_NODES = 10000
N_EDGES = 320000
D = 128
DJ = D // L

OUT_LO = 1000
OUT_N = 5000

EPW = N_EDGES // NW
CHUNK = 80
NCHUNK = EPW // CHUNK
GPC = CHUNK // L

ACC_ROWS = 5120
RPT = ACC_ROWS // NS

_DNUMS = lax.GatherDimensionNumbers(
    offset_dims=(), collapsed_slice_dims=(0,), start_index_map=(0,))


def _lane_perm(vec, idx16):
    return lax.gather(vec, idx16.reshape(L, 1), _DNUMS, slice_sizes=(1,),
                      mode=lax.GatherScatterMode.PROMISE_IN_BOUNDS)


def _lane_bcast(vec, l):
    return _lane_perm(vec, jnp.full((L,), l, jnp.int32))


def _lane_sum(vec, iota16):
    for sh in (8, 4, 2, 1):
        vec = vec + _lane_perm(vec, jnp.bitwise_xor(iota16, sh))
    return vec


def _prefix_sum(vec, iota16):
    zero = jnp.zeros((L,), jnp.int32)
    for sh in (1, 2, 4, 8):
        shifted = _lane_perm(vec, jnp.maximum(iota16 - sh, 0))
        vec = vec + jnp.where(iota16 >= sh, shifted, zero)
    return vec


_mesh = plsc.VectorSubcoreMesh(
    core_axis_name="c", subcore_axis_name="s", num_cores=NC, num_subcores=NS
)


NPT = 640
NPAD = NS * NPT


def _sc_body(interval_hbm, emb_hbm, row_hbm, col_hbm, params_hbm,
             out_hbm, theta_hbm,
             col_all, row_all, int_all, params_v, th0, th1,
             in0, in1, out0, out1, dst0, dst1,
             acc_sh, gsem0, gsem1, ssem0, ssem1):
    c = lax.axis_index("c")
    s = lax.axis_index("s")
    wid = s * NC + c
    iota16 = lax.iota(jnp.int32, L)
    ins = (in0, in1)
    outs = (out0, out1)
    dsts = (dst0, dst1)
    ths = (th0, th1)
    gsems = (gsem0, gsem1)
    ssems = (ssem0, ssem1)

    def _zb_zero(r, _):
        for j in range(DJ):
            out0[r, pl.ds(j * L, L)] = jnp.zeros((L,), jnp.float32)
        return 0

    lax.fori_loop(0, CHUNK, _zb_zero, 0)
    off0 = s * RPT
    for q in range(RPT // CHUNK):
        pltpu.sync_copy(out0, acc_sh.at[pl.ds(off0 + q * CHUNK, CHUNK)])

    ebase = wid * EPW
    pltpu.sync_copy(col_hbm.at[pl.ds(ebase, EPW)], col_all.at[pl.ds(0, EPW)])
    pltpu.sync_copy(row_hbm.at[pl.ds(ebase, EPW)], row_all.at[pl.ds(0, EPW)])
    pltpu.sync_copy(interval_hbm.at[pl.ds(ebase, EPW)],
                    int_all.at[pl.ds(0, EPW)])
    pltpu.sync_copy(params_hbm, params_v)

    nb = s * NPT
    p = [params_v[pl.ds(j * L, L)] for j in range(DJ)]

    def _tchunk(q, _):
        for g in range(GPC):
            dst0[pl.ds(g * L, L)] = jnp.minimum(
                nb + q * CHUNK + g * L + iota16, N_NODES - 1)
        pltpu.async_copy(emb_hbm.at[dst0], in0, gsem0).wait()

        def _trow(g, _):
            th16 = jnp.zeros((L,), jnp.float32)
            for l in range(L):
                r = g * L + l
                acc16 = in0[r, pl.ds(0, L)] * p[0]
                for j in range(1, DJ):
                    acc16 = acc16 + in0[r, pl.ds(j * L, L)] * p[j]
                th = _lane_sum(acc16, iota16)
                th16 = jnp.where(iota16 == l, th, th16)
            th0[pl.ds(g * L, L)] = th16
            return 0

        lax.fori_loop(0, GPC, _trow, 0)
        pltpu.sync_copy(th0, theta_hbm.at[pl.ds(nb + q * CHUNK, CHUNK)])
        return 0

    lax.fori_loop(0, NPT // CHUNK, _tchunk, 0)

    def _filt(i, cnt):
        off = i * L
        col16 = col_all[pl.ds(off, L)]
        row16 = row_all[pl.ds(off, L)]
        int16 = int_all[pl.ds(off, L)]
        ok = (row16 >= OUT_LO) & (row16 < OUT_LO + OUT_N)
        incl = _prefix_sum(jnp.where(ok, 1, 0), iota16)
        target = iota16 + 1
        pos = jnp.zeros((L,), jnp.int32)
        for sh in (8, 4, 2, 1):
            vals = _lane_perm(incl, pos + (sh - 1))
            pos = jnp.where(vals < target, pos + sh, pos)
        sel = jnp.minimum(pos, L - 1)
        col_all[pl.ds(cnt, L)] = _lane_perm(col16, sel)
        int_all[pl.ds(cnt, L)] = _lane_perm(int16, sel)
        row_all[pl.ds(cnt, L)] = _lane_perm(row16 - OUT_LO, sel)
        pop = incl[L - 1]
        return cnt + pop

    cnt_s = lax.fori_loop(0, EPW // L, _filt, 0)

    for g in range(2 * CHUNK // L):
        col_all[pl.ds(cnt_s + g * L, L)] = wid * 300 + g * L + iota16
        int_all[pl.ds(cnt_s + g * L, L)] = jnp.zeros((L,), jnp.float32)
        row_all[pl.ds(cnt_s + g * L, L)] = OUT_N + iota16

    npairs = jnp.maximum((cnt_s + 2 * CHUNK - 1) // (2 * CHUNK), 1)

    plsc.subcore_barrier()

    def _gather_start(ci, b):
        idx = col_all.at[pl.ds(ci * CHUNK, CHUNK)]
        pltpu.async_copy(emb_hbm.at[idx], ins[b], gsems[b])
        pltpu.async_copy(theta_hbm.at[idx], ths[b], gsems[b])

    def _gather_wait(b):
        idx = col_all.at[pl.ds(0, CHUNK)]
        pltpu.make_async_copy(emb_hbm.at[idx], ins[b], gsems[b]).wait()
        pltpu.make_async_copy(theta_hbm.at[idx], ths[b], gsems[b]).wait()

    def _scatter_start(b):
        pltpu.async_copy(outs[b], acc_sh.at[dsts[b]], ssems[b], add=True)

    def _scatter_wait(b):
        pltpu.make_async_copy(outs[b], acc_sh.at[dsts[b]], ssems[b]).wait()

    def _compute(ci, b):
        base = ci * CHUNK
        ib, ob, db = ins[b], outs[b], dsts[b]

        def _group(g, _):
            off = base + g * L
            int16 = int_all[pl.ds(off, L)]
            db[pl.ds(g * L, L)] = row_all[pl.ds(off, L)]
            th16 = ths[b][pl.ds(g * L, L)]
            d16 = jnp.exp(int16 * th16)
            for l in range(L):
                e = g * L + l
                dl = _lane_bcast(d16, l)
                for j in range(DJ):
                    ob[e, pl.ds(j * L, L)] = ib[e, pl.ds(j * L, L)] * dl
            return 0

        lax.fori_loop(0, GPC, _group, 0)

    for b in range(2):
        for g in range(GPC):
            dsts[b][pl.ds(g * L, L)] = OUT_N + iota16
        _scatter_start(b)
    _gather_start(0, 0)
    _gather_start(1, 1)

    def _pair(k, _):
        for b in range(2):
            ci = 2 * k + b
            _gather_wait(b)
            _scatter_wait(b)
            _compute(ci, b)
            _scatter_start(b)

            @pl.when(ci + 2 < 2 * npairs)
            def _():
                _gather_start(ci + 2, b)
        return 0

    lax.fori_loop(0, npairs, _pair, 0)

    _scatter_wait(0)
    _scatter_wait(1)

    plsc.subcore_barrier()

    for q in range(RPT // CHUNK):
        pltpu.sync_copy(acc_sh.at[pl.ds(off0 + q * CHUNK, CHUNK)], in0)
        pltpu.sync_copy(in0, out_hbm.at[c, pl.ds(off0 + q * CHUNK, CHUNK)])


_sc_kernel = pl.kernel(
    _sc_body,
    out_type=(jax.ShapeDtypeStruct((NC, ACC_ROWS, D), jnp.float32),
              jax.ShapeDtypeStruct((NPAD,), jnp.float32)),
    mesh=_mesh,
    scratch_types=[
        pltpu.VMEM((EPW + 2 * CHUNK,), jnp.int32),
        pltpu.VMEM((EPW + 2 * CHUNK,), jnp.int32),
        pltpu.VMEM((EPW + 2 * CHUNK,), jnp.float32),
        pltpu.VMEM((D,), jnp.float32),
        pltpu.VMEM((CHUNK,), jnp.float32),
        pltpu.VMEM((CHUNK,), jnp.float32),
        pltpu.VMEM((CHUNK, D), jnp.float32),
        pltpu.VMEM((CHUNK, D), jnp.float32),
        pltpu.VMEM((CHUNK, D), jnp.float32),
        pltpu.VMEM((CHUNK, D), jnp.float32),
        pltpu.VMEM((CHUNK,), jnp.int32),
        pltpu.VMEM((CHUNK,), jnp.int32),
        pltpu.VMEM_SHARED((ACC_ROWS, D), jnp.float32),
        pltpu.SemaphoreType.DMA,
        pltpu.SemaphoreType.DMA,
        pltpu.SemaphoreType.DMA,
        pltpu.SemaphoreType.DMA,
    ],
)


def _combine_body(p_ref, o_ref):
    o_ref[...] = p_ref[0] + p_ref[1]


_combine = pl.pallas_call(
    _combine_body,
    out_shape=jax.ShapeDtypeStruct((OUT_N, D), jnp.float32),
    grid=(5,),
    in_specs=[pl.BlockSpec((2, 1000, D), lambda i: (0, i, 0))],
    out_specs=pl.BlockSpec((1000, D), lambda i: (i, 0)),
)


def kernel(interval, embedding, edge_index, params):
    row = edge_index[0]
    col = edge_index[1]
    partial, _theta = _sc_kernel(interval, embedding, row, col,
                                 params.reshape(D))
    return _combine(partial)


# speedup vs baseline: 1.4203x; 1.0475x over previous
import jax
import jax.numpy as jnp
from jax import lax
from jax.experimental import pallas as pl
from jax.experimental.pallas import tpu as pltpu
from jax.experimental.pallas import tpu_sc as plsc

NC = 2
NS = 16
L = 16
NW = NC * NS

N_NODES = 10000
N_EDGES = 320000
D = 128
DJ = D // L

OUT_LO = 1000
OUT_N = 5000

EPW = N_EDGES // NW
CHUNK = 80
NCHUNK = EPW // CHUNK
GPC = CHUNK // L

ACC_ROWS = 5120
RPT = ACC_ROWS // NS

_DNUMS = lax.GatherDimensionNumbers(
    offset_dims=(), collapsed_slice_dims=(0,), start_index_map=(0,))


def _lane_perm(vec, idx16):
    return lax.gather(vec, idx16.reshape(L, 1), _DNUMS, slice_sizes=(1,),
                      mode=lax.GatherScatterMode.PROMISE_IN_BOUNDS)


def _lane_bcast(vec, l):
    return _lane_perm(vec, jnp.full((L,), l, jnp.int32))


def _lane_sum(vec, iota16):
    for sh in (8, 4, 2, 1):
        vec = vec + _lane_perm(vec, jnp.bitwise_xor(iota16, sh))
    return vec


def _prefix_sum(vec, iota16):
    zero = jnp.zeros((L,), jnp.int32)
    for sh in (1, 2, 4, 8):
        shifted = _lane_perm(vec, jnp.maximum(iota16 - sh, 0))
        vec = vec + jnp.where(iota16 >= sh, shifted, zero)
    return vec


_mesh = plsc.VectorSubcoreMesh(
    core_axis_name="c", subcore_axis_name="s", num_cores=NC, num_subcores=NS
)


NPT = 640
NPAD = NS * NPT


def _sc_body(interval_hbm, emb_hbm, row_hbm, col_hbm, params_hbm,
             out_hbm, theta_hbm,
             col_all, row_all, int_all, params_v, th0, th1,
             in0, in1, out0, out1, dst0, dst1,
             acc_sh, gsem0, gsem1, ssem0, ssem1):
    c = lax.axis_index("c")
    s = lax.axis_index("s")
    wid = s * NC + c
    iota16 = lax.iota(jnp.int32, L)
    ins = (in0, in1)
    outs = (out0, out1)
    dsts = (dst0, dst1)
    ths = (th0, th1)
    gsems = (gsem0, gsem1)
    ssems = (ssem0, ssem1)

    ebase = wid * EPW
    pltpu.async_copy(col_hbm.at[pl.ds(ebase, EPW)],
                     col_all.at[pl.ds(0, EPW)], ssem0)
    pltpu.async_copy(row_hbm.at[pl.ds(ebase, EPW)],
                     row_all.at[pl.ds(0, EPW)], ssem0)
    pltpu.async_copy(interval_hbm.at[pl.ds(ebase, EPW)],
                     int_all.at[pl.ds(0, EPW)], ssem1)
    pltpu.async_copy(params_hbm, params_v, ssem1)

    def _zb_zero(r, _):
        for j in range(DJ):
            out0[r, pl.ds(j * L, L)] = jnp.zeros((L,), jnp.float32)
        return 0

    lax.fori_loop(0, CHUNK, _zb_zero, 0)
    off0 = s * RPT
    for q in range(RPT // CHUNK):
        pltpu.sync_copy(out0, acc_sh.at[pl.ds(off0 + q * CHUNK, CHUNK)])

    pltpu.make_async_copy(col_hbm.at[pl.ds(ebase, EPW)],
                          col_all.at[pl.ds(0, EPW)], ssem0).wait()
    pltpu.make_async_copy(row_hbm.at[pl.ds(ebase, EPW)],
                          row_all.at[pl.ds(0, EPW)], ssem0).wait()
    pltpu.make_async_copy(interval_hbm.at[pl.ds(ebase, EPW)],
                          int_all.at[pl.ds(0, EPW)], ssem1).wait()
    pltpu.make_async_copy(params_hbm, params_v, ssem1).wait()

    nb = s * NPT
    p = [params_v[pl.ds(j * L, L)] for j in range(DJ)]
    NTCH = NPT // CHUNK

    def _tidx(q, db):
        for g in range(GPC):
            db[pl.ds(g * L, L)] = jnp.minimum(
                nb + q * CHUNK + g * L + iota16, N_NODES - 1)

    def _tg_start(b):
        pltpu.async_copy(emb_hbm.at[dsts[b]], ins[b], gsems[b])

    def _tg_wait(b):
        pltpu.make_async_copy(emb_hbm.at[dsts[b]], ins[b], gsems[b]).wait()

    _tidx(0, dst0)
    _tg_start(0)
    _tidx(1, dst1)
    _tg_start(1)

    def _tpair(k, _):
        for b in range(2):
            q = 2 * k + b
            _tg_wait(b)

            def _trow(g, _):
                th16 = jnp.zeros((L,), jnp.float32)
                for l in range(L):
                    r = g * L + l
                    acc16 = ins[b][r, pl.ds(0, L)] * p[0]
                    for j in range(1, DJ):
                        acc16 = acc16 + ins[b][r, pl.ds(j * L, L)] * p[j]
                    th = _lane_sum(acc16, iota16)
                    th16 = jnp.where(iota16 == l, th, th16)
                th0[pl.ds(g * L, L)] = th16
                return 0

            lax.fori_loop(0, GPC, _trow, 0)
            pltpu.sync_copy(th0, theta_hbm.at[pl.ds(nb + q * CHUNK, CHUNK)])

            @pl.when(q + 2 < NTCH)
            def _():
                _tidx(q + 2, dsts[b])
                _tg_start(b)
        return 0

    lax.fori_loop(0, NTCH // 2, _tpair, 0)

    def _filt(i, cnt):
        off = i * L
        col16 = col_all[pl.ds(off, L)]
        row16 = row_all[pl.ds(off, L)]
        int16 = int_all[pl.ds(off, L)]
        ok = (row16 >= OUT_LO) & (row16 < OUT_LO + OUT_N)
        incl = _prefix_sum(jnp.where(ok, 1, 0), iota16)
        target = iota16 + 1
        pos = jnp.zeros((L,), jnp.int32)
        for sh in (8, 4, 2, 1):
            vals = _lane_perm(incl, pos + (sh - 1))
            pos = jnp.where(vals < target, pos + sh, pos)
        sel = jnp.minimum(pos, L - 1)
        col_all[pl.ds(cnt, L)] = _lane_perm(col16, sel)
        int_all[pl.ds(cnt, L)] = _lane_perm(int16, sel)
        row_all[pl.ds(cnt, L)] = _lane_perm(row16 - OUT_LO, sel)
        pop = incl[L - 1]
        return cnt + pop

    cnt_s = lax.fori_loop(0, EPW // L, _filt, 0)

    for g in range(2 * CHUNK // L):
        col_all[pl.ds(cnt_s + g * L, L)] = wid * 300 + g * L + iota16
        int_all[pl.ds(cnt_s + g * L, L)] = jnp.zeros((L,), jnp.float32)
        row_all[pl.ds(cnt_s + g * L, L)] = OUT_N + iota16

    npairs = jnp.maximum((cnt_s + 2 * CHUNK - 1) // (2 * CHUNK), 1)

    plsc.subcore_barrier()

    def _gather_start(ci, b):
        idx = col_all.at[pl.ds(ci * CHUNK, CHUNK)]
        pltpu.async_copy(emb_hbm.at[idx], ins[b], gsems[b])
        pltpu.async_copy(theta_hbm.at[idx], ths[b], gsems[b])

    def _gather_wait(b):
        idx = col_all.at[pl.ds(0, CHUNK)]
        pltpu.make_async_copy(emb_hbm.at[idx], ins[b], gsems[b]).wait()
        pltpu.make_async_copy(theta_hbm.at[idx], ths[b], gsems[b]).wait()

    def _scatter_start(b):
        pltpu.async_copy(outs[b], acc_sh.at[dsts[b]], ssems[b], add=True)

    def _scatter_wait(b):
        pltpu.make_async_copy(outs[b], acc_sh.at[dsts[b]], ssems[b]).wait()

    def _compute(ci, b):
        base = ci * CHUNK
        ib, ob, db = ins[b], outs[b], dsts[b]

        def _group(g, _):
            off = base + g * L
            int16 = int_all[pl.ds(off, L)]
            db[pl.ds(g * L, L)] = row_all[pl.ds(off, L)]
            th16 = ths[b][pl.ds(g * L, L)]
            d16 = jnp.exp(int16 * th16)
            for l in range(L):
                e = g * L + l
                dl = _lane_bcast(d16, l)
                for j in range(DJ):
                    ob[e, pl.ds(j * L, L)] = ib[e, pl.ds(j * L, L)] * dl
            return 0

        lax.fori_loop(0, GPC, _group, 0)

    for b in range(2):
        for g in range(GPC):
            dsts[b][pl.ds(g * L, L)] = OUT_N + iota16
        _scatter_start(b)
    _gather_start(0, 0)
    _gather_start(1, 1)

    def _pair(k, _):
        for b in range(2):
            ci = 2 * k + b
            _gather_wait(b)
            _scatter_wait(b)
            _compute(ci, b)
            _scatter_start(b)

            @pl.when(ci + 2 < 2 * npairs)
            def _():
                _gather_start(ci + 2, b)
        return 0

    lax.fori_loop(0, npairs, _pair, 0)

    _scatter_wait(0)
    _scatter_wait(1)

    plsc.subcore_barrier()

    for q in range(RPT // CHUNK):
        pltpu.sync_copy(acc_sh.at[pl.ds(off0 + q * CHUNK, CHUNK)], in0)
        pltpu.sync_copy(in0, out_hbm.at[c, pl.ds(off0 + q * CHUNK, CHUNK)])


_sc_kernel = pl.kernel(
    _sc_body,
    out_type=(jax.ShapeDtypeStruct((NC, ACC_ROWS, D), jnp.float32),
              jax.ShapeDtypeStruct((NPAD,), jnp.float32)),
    mesh=_mesh,
    scratch_types=[
        pltpu.VMEM((EPW + 2 * CHUNK,), jnp.int32),
        pltpu.VMEM((EPW + 2 * CHUNK,), jnp.int32),
        pltpu.VMEM((EPW + 2 * CHUNK,), jnp.float32),
        pltpu.VMEM((D,), jnp.float32),
        pltpu.VMEM((CHUNK,), jnp.float32),
        pltpu.VMEM((CHUNK,), jnp.float32),
        pltpu.VMEM((CHUNK, D), jnp.float32),
        pltpu.VMEM((CHUNK, D), jnp.float32),
        pltpu.VMEM((CHUNK, D), jnp.float32),
        pltpu.VMEM((CHUNK, D), jnp.float32),
        pltpu.VMEM((CHUNK,), jnp.int32),
        pltpu.VMEM((CHUNK,), jnp.int32),
        pltpu.VMEM_SHARED((ACC_ROWS, D), jnp.float32),
        pltpu.SemaphoreType.DMA,
        pltpu.SemaphoreType.DMA,
        pltpu.SemaphoreType.DMA,
        pltpu.SemaphoreType.DMA,
    ],
)


def _combine_body(p_ref, o_ref):
    o_ref[...] = p_ref[0] + p_ref[1]


_combine = pl.pallas_call(
    _combine_body,
    out_shape=jax.ShapeDtypeStruct((OUT_N, D), jnp.float32),
    grid=(5,),
    in_specs=[pl.BlockSpec((2, 1000, D), lambda i: (0, i, 0))],
    out_specs=pl.BlockSpec((1000, D), lambda i: (i, 0)),
)


def kernel(interval, embedding, edge_index, params):
    row = edge_index[0]
    col = edge_index[1]
    partial, _theta = _sc_kernel(interval, embedding, row, col,
                                 params.reshape(D))
    return _combine(partial)


# speedup vs baseline: 1.4289x; 1.0061x over previous
import jax
import jax.numpy as jnp
from jax import lax
from jax.experimental import pallas as pl
from jax.experimental.pallas import tpu as pltpu
from jax.experimental.pallas import tpu_sc as plsc

NC = 2
NS = 16
L = 16
NW = NC * NS

N_NODES = 10000
N_EDGES = 320000
D = 128
DJ = D // L

OUT_LO = 1000
OUT_N = 5000

EPW = N_EDGES // NW
CHUNK = 80
NCHUNK = EPW // CHUNK
GPC = CHUNK // L

ACC_ROWS = 5120
RPT = ACC_ROWS // NS

_DNUMS = lax.GatherDimensionNumbers(
    offset_dims=(), collapsed_slice_dims=(0,), start_index_map=(0,))


def _lane_perm(vec, idx16):
    return lax.gather(vec, idx16.reshape(L, 1), _DNUMS, slice_sizes=(1,),
                      mode=lax.GatherScatterMode.PROMISE_IN_BOUNDS)


def _lane_bcast(vec, l):
    return _lane_perm(vec, jnp.full((L,), l, jnp.int32))


def _lane_sum(vec, iota16):
    for sh in (8, 4, 2, 1):
        vec = vec + _lane_perm(vec, jnp.bitwise_xor(iota16, sh))
    return vec


def _prefix_sum(vec, iota16):
    zero = jnp.zeros((L,), jnp.int32)
    for sh in (1, 2, 4, 8):
        shifted = _lane_perm(vec, jnp.maximum(iota16 - sh, 0))
        vec = vec + jnp.where(iota16 >= sh, shifted, zero)
    return vec


_mesh = plsc.VectorSubcoreMesh(
    core_axis_name="c", subcore_axis_name="s", num_cores=NC, num_subcores=NS
)


NPT = 640
NPAD = NS * NPT


def _sc_body(interval_hbm, emb_hbm, row_hbm, col_hbm, params_hbm,
             out_hbm, theta_hbm,
             col_all, row_all, int_all, params_v, th0, th1,
             in0, in1, out0, out1, dst0, dst1,
             acc_sh, gsem0, gsem1, ssem0, ssem1):
    c = lax.axis_index("c")
    s = lax.axis_index("s")
    wid = s * NC + c
    iota16 = lax.iota(jnp.int32, L)
    ins = (in0, in1)
    outs = (out0, out1)
    dsts = (dst0, dst1)
    ths = (th0, th1)
    gsems = (gsem0, gsem1)
    ssems = (ssem0, ssem1)

    ebase = wid * EPW
    pltpu.async_copy(col_hbm.at[pl.ds(ebase, EPW)],
                     col_all.at[pl.ds(0, EPW)], ssem0)
    pltpu.async_copy(row_hbm.at[pl.ds(ebase, EPW)],
                     row_all.at[pl.ds(0, EPW)], ssem0)
    pltpu.async_copy(interval_hbm.at[pl.ds(ebase, EPW)],
                     int_all.at[pl.ds(0, EPW)], ssem1)
    pltpu.async_copy(params_hbm, params_v, ssem1)

    def _zb_zero(r, _):
        for j in range(DJ):
            out0[r, pl.ds(j * L, L)] = jnp.zeros((L,), jnp.float32)
        return 0

    lax.fori_loop(0, CHUNK, _zb_zero, 0)
    off0 = s * RPT
    for q in range(RPT // CHUNK):
        pltpu.sync_copy(out0, acc_sh.at[pl.ds(off0 + q * CHUNK, CHUNK)])

    pltpu.make_async_copy(col_hbm.at[pl.ds(ebase, EPW)],
                          col_all.at[pl.ds(0, EPW)], ssem0).wait()
    pltpu.make_async_copy(row_hbm.at[pl.ds(ebase, EPW)],
                          row_all.at[pl.ds(0, EPW)], ssem0).wait()
    pltpu.make_async_copy(interval_hbm.at[pl.ds(ebase, EPW)],
                          int_all.at[pl.ds(0, EPW)], ssem1).wait()
    pltpu.make_async_copy(params_hbm, params_v, ssem1).wait()

    nb = s * NPT
    p = [params_v[pl.ds(j * L, L)] for j in range(DJ)]
    NTCH = NPT // CHUNK

    def _tidx(q, db):
        for g in range(GPC):
            db[pl.ds(g * L, L)] = jnp.minimum(
                nb + q * CHUNK + g * L + iota16, N_NODES - 1)

    def _tg_start(b):
        pltpu.async_copy(emb_hbm.at[dsts[b]], ins[b], gsems[b])

    def _tg_wait(b):
        pltpu.make_async_copy(emb_hbm.at[dsts[b]], ins[b], gsems[b]).wait()

    _tidx(0, dst0)
    _tg_start(0)
    _tidx(1, dst1)
    _tg_start(1)

    def _tpair(k, _):
        for b in range(2):
            q = 2 * k + b
            _tg_wait(b)

            def _trow(g, _):
                th16 = jnp.zeros((L,), jnp.float32)
                for l in range(L):
                    r = g * L + l
                    acc16 = ins[b][r, pl.ds(0, L)] * p[0]
                    for j in range(1, DJ):
                        acc16 = acc16 + ins[b][r, pl.ds(j * L, L)] * p[j]
                    th = _lane_sum(acc16, iota16)
                    th16 = jnp.where(iota16 == l, th, th16)
                th0[pl.ds(g * L, L)] = th16
                return 0

            lax.fori_loop(0, GPC, _trow, 0)
            pltpu.sync_copy(th0, theta_hbm.at[pl.ds(nb + q * CHUNK, CHUNK)])

            @pl.when(q + 2 < NTCH)
            def _():
                _tidx(q + 2, dsts[b])
                _tg_start(b)
        return 0

    lax.fori_loop(0, NTCH // 2, _tpair, 0)

    def _filt(i, cnt):
        off = i * L
        col16 = col_all[pl.ds(off, L)]
        row16 = row_all[pl.ds(off, L)]
        int16 = int_all[pl.ds(off, L)]
        ok = (row16 >= OUT_LO) & (row16 < OUT_LO + OUT_N)
        incl = _prefix_sum(jnp.where(ok, 1, 0), iota16)
        target = iota16 + 1
        pos = jnp.zeros((L,), jnp.int32)
        for sh in (8, 4, 2, 1):
            vals = _lane_perm(incl, pos + (sh - 1))
            pos = jnp.where(vals < target, pos + sh, pos)
        sel = jnp.minimum(pos, L - 1)
        col_all[pl.ds(cnt, L)] = _lane_perm(col16, sel)
        int_all[pl.ds(cnt, L)] = _lane_perm(int16, sel)
        row_all[pl.ds(cnt, L)] = _lane_perm(row16 - OUT_LO, sel)
        pop = incl[L - 1]
        return cnt + pop

    cnt_s = lax.fori_loop(0, EPW // L, _filt, 0)

    for g in range(2 * CHUNK // L):
        col_all[pl.ds(cnt_s + g * L, L)] = wid * 300 + g * L + iota16
        int_all[pl.ds(cnt_s + g * L, L)] = jnp.zeros((L,), jnp.float32)
        row_all[pl.ds(cnt_s + g * L, L)] = OUT_N + iota16

    npairs = jnp.maximum((cnt_s + 2 * CHUNK - 1) // (2 * CHUNK), 1)

    plsc.subcore_barrier()

    def _gather_start(ci, b):
        idx = col_all.at[pl.ds(ci * CHUNK, CHUNK)]
        pltpu.async_copy(emb_hbm.at[idx], ins[b], gsems[b])
        pltpu.async_copy(theta_hbm.at[idx], ths[b], gsems[b])

    def _gather_wait(b):
        idx = col_all.at[pl.ds(0, CHUNK)]
        pltpu.make_async_copy(emb_hbm.at[idx], ins[b], gsems[b]).wait()
        pltpu.make_async_copy(theta_hbm.at[idx], ths[b], gsems[b]).wait()

    def _scatter_start(b):
        pltpu.async_copy(outs[b], acc_sh.at[dsts[b]], ssems[b], add=True)

    def _scatter_wait(b):
        pltpu.make_async_copy(outs[b], acc_sh.at[dsts[b]], ssems[b]).wait()

    def _compute(ci, b):
        base = ci * CHUNK
        ib, ob, db = ins[b], outs[b], dsts[b]

        def _group(g, _):
            off = base + g * L
            int16 = int_all[pl.ds(off, L)]
            db[pl.ds(g * L, L)] = row_all[pl.ds(off, L)]
            th16 = ths[b][pl.ds(g * L, L)]
            d16 = jnp.exp(int16 * th16)
            for l in range(L):
                e = g * L + l
                dl = _lane_bcast(d16, l)
                for j in range(DJ):
                    ob[e, pl.ds(j * L, L)] = ib[e, pl.ds(j * L, L)] * dl
            return 0

        lax.fori_loop(0, GPC, _group, 0)

    for b in range(2):
        for g in range(GPC):
            dsts[b][pl.ds(g * L, L)] = OUT_N + iota16
        _scatter_start(b)
    _gather_start(0, 0)
    _gather_start(1, 1)

    def _pair(k, _):
        for b in range(2):
            ci = 2 * k + b
            _gather_wait(b)
            _scatter_wait(b)
            _compute(ci, b)
            _scatter_start(b)

            @pl.when(ci + 2 < 2 * npairs)
            def _():
                _gather_start(ci + 2, b)
        return 0

    lax.fori_loop(0, npairs, _pair, 0)

    _scatter_wait(0)
    _scatter_wait(1)

    plsc.subcore_barrier()

    for q in range(RPT // CHUNK):
        b = q % 2
        if q >= 2:
            pltpu.make_async_copy(
                ins[b], out_hbm.at[c, pl.ds(off0 + q * CHUNK, CHUNK)],
                gsems[b]).wait()
        pltpu.sync_copy(acc_sh.at[pl.ds(off0 + q * CHUNK, CHUNK)], ins[b])
        pltpu.async_copy(ins[b], out_hbm.at[c, pl.ds(off0 + q * CHUNK, CHUNK)],
                         gsems[b])
    for b in range(2):
        pltpu.make_async_copy(
            ins[b], out_hbm.at[c, pl.ds(off0 + (2 + b) * CHUNK, CHUNK)],
            gsems[b]).wait()


_sc_kernel = pl.kernel(
    _sc_body,
    out_type=(jax.ShapeDtypeStruct((NC, ACC_ROWS, D), jnp.float32),
              jax.ShapeDtypeStruct((NPAD,), jnp.float32)),
    mesh=_mesh,
    scratch_types=[
        pltpu.VMEM((EPW + 2 * CHUNK,), jnp.int32),
        pltpu.VMEM((EPW + 2 * CHUNK,), jnp.int32),
        pltpu.VMEM((EPW + 2 * CHUNK,), jnp.float32),
        pltpu.VMEM((D,), jnp.float32),
        pltpu.VMEM((CHUNK,), jnp.float32),
        pltpu.VMEM((CHUNK,), jnp.float32),
        pltpu.VMEM((CHUNK, D), jnp.float32),
        pltpu.VMEM((CHUNK, D), jnp.float32),
        pltpu.VMEM((CHUNK, D), jnp.float32),
        pltpu.VMEM((CHUNK, D), jnp.float32),
        pltpu.VMEM((CHUNK,), jnp.int32),
        pltpu.VMEM((CHUNK,), jnp.int32),
        pltpu.VMEM_SHARED((ACC_ROWS, D), jnp.float32),
        pltpu.SemaphoreType.DMA,
        pltpu.SemaphoreType.DMA,
        pltpu.SemaphoreType.DMA,
        pltpu.SemaphoreType.DMA,
    ],
)


def _combine_body(p_ref, o_ref):
    o_ref[...] = p_ref[0] + p_ref[1]


_combine = pl.pallas_call(
    _combine_body,
    out_shape=jax.ShapeDtypeStruct((OUT_N, D), jnp.float32),
    grid=(5,),
    in_specs=[pl.BlockSpec((2, 1000, D), lambda i: (0, i, 0))],
    out_specs=pl.BlockSpec((1000, D), lambda i: (i, 0)),
)


def kernel(interval, embedding, edge_index, params):
    row = edge_index[0]
    col = edge_index[1]
    partial, _theta = _sc_kernel(interval, embedding, row, col,
                                 params.reshape(D))
    return _combine(partial)
